# baseline (speedup 1.0000x reference)
import functools

import jax
import jax.numpy as jnp
from jax import lax
from jax.experimental import pallas as pl
from jax.experimental.pallas import tpu as pltpu
from jax.experimental.pallas import tpu_sc as plsc

N = 10000
E = 320000
EPS = 1e-5

_NC = 2
_NS = 16
_P = 10112
_SLAB = _P // _NS
_DUMP = _P - 1
_EPAD = 327680

_CHD = 128
_NCHD = 80
_NBD = 5

_CHA = 64
_PCH = 40
_NBA = 4
_PH_HEAVY = 8
_PH_LIGHT = 0
_NE_HEAVY = _NS * _PH_HEAVY * _PCH * _CHA
_NE_LIGHT = _NS * _PH_LIGHT * _PCH * _CHA

_BM = 400
_GRID = N // _BM

_mesh = plsc.VectorSubcoreMesh(core_axis_name="c", subcore_axis_name="s")


@functools.partial(
    pl.kernel,
    out_type=jax.ShapeDtypeStruct((_NC, _P, 128), jnp.float32),
    mesh=_mesh,
    scratch_types=[
        pltpu.VMEM_SHARED((_P, 128), jnp.float32),
        pltpu.VMEM((_NCHD, _CHD), jnp.int32),
        pltpu.VMEM((_CHD, 128), jnp.float32),
        pltpu.SemaphoreType.DMA,
    ],
)
def _sc_degree(dst_hbm, ones_hbm, z128_hbm, out_hbm, sh, dsti, onesb, sem):
    c = lax.axis_index("c")
    s = lax.axis_index("s")
    lo = s * _SLAB
    pltpu.sync_copy(dst_hbm.at[c, s], dsti)
    pltpu.sync_copy(z128_hbm, sh.at[pl.ds(lo, _SLAB)])
    pltpu.sync_copy(ones_hbm, onesb)
    plsc.subcore_barrier()

    def body(g, carry):
        base = g * _NBD
        for b in range(_NBD):
            pltpu.async_copy(onesb, sh.at[dsti.at[base + b]], sem, add=True)
        for b in range(_NBD):
            pltpu.make_async_copy(onesb, sh.at[dsti.at[base + b]], sem).wait()
        return carry

    lax.fori_loop(0, _NCHD // _NBD, body, 0)
    plsc.subcore_barrier()
    pltpu.sync_copy(sh.at[pl.ds(lo, _SLAB)], out_hbm.at[c, pl.ds(lo, _SLAB)])


@functools.partial(
    pl.kernel,
    out_type=jax.ShapeDtypeStruct((_NC, _P, 128), jnp.float32),
    mesh=_mesh,
    scratch_types=[
        pltpu.VMEM_SHARED((_P, 128), jnp.float32),
        pltpu.VMEM((_PCH, _CHA), jnp.int32),
        pltpu.VMEM((_PCH, _CHA), jnp.int32),
        pltpu.VMEM((_NBA, _CHA, 128), jnp.float32),
        pltpu.SemaphoreType.DMA((_NBA,)),
        pltpu.SemaphoreType.DMA((_NBA,)),
    ],
)
def _sc_aggregate(x_hbm, src_hbm, dst_hbm, z128_hbm, out_hbm, sh,
                  srci, dsti, rows, sem_g, sem_s):
    c = lax.axis_index("c")
    s = lax.axis_index("s")
    lo = s * _SLAB
    nph = lax.select(c == 0, _PH_LIGHT, _PH_HEAVY)
    pltpu.sync_copy(z128_hbm, sh.at[pl.ds(lo, _SLAB)])
    plsc.subcore_barrier()

    def phase(p, carry):
        @pl.when(p < nph)
        def _():
            pltpu.sync_copy(src_hbm.at[c, s, pl.ds(p * _PCH, _PCH)], srci)
            pltpu.sync_copy(dst_hbm.at[c, s, pl.ds(p * _PCH, _PCH)], dsti)

            def rnd(r, cc):
                base = r * _NBA
                for b in range(_NBA):
                    @pl.when(r > 0)
                    def _():
                        pltpu.make_async_copy(
                            rows.at[b], sh.at[dsti.at[base - _NBA + b]],
                            sem_s.at[b]).wait()
                    pltpu.async_copy(x_hbm.at[srci.at[base + b]], rows.at[b],
                                     sem_g.at[b])
                for b in range(_NBA):
                    pltpu.make_async_copy(
                        x_hbm.at[srci.at[base + b]], rows.at[b],
                        sem_g.at[b]).wait()
                    pltpu.async_copy(rows.at[b], sh.at[dsti.at[base + b]],
                                     sem_s.at[b], add=True)
                return cc

            lax.fori_loop(0, _PCH // _NBA, rnd, 0)
            for b in range(_NBA):
                pltpu.make_async_copy(
                    rows.at[b], sh.at[dsti.at[_PCH - _NBA + b]],
                    sem_s.at[b]).wait()

        return carry

    lax.fori_loop(0, _PH_HEAVY, phase, 0)
    plsc.subcore_barrier()
    pltpu.sync_copy(sh.at[pl.ds(lo, _SLAB)], out_hbm.at[c, pl.ds(lo, _SLAB)])


def _dinv_block(d0, d1):
    deg = d0[0, :, :1] + d1[0, :, :1] + 1.0
    return lax.rsqrt(deg)


def _prescale_kernel(us_ref, uy_ref, d0_ref, d1_ref, ls_ref):
    dinv = _dinv_block(d0_ref, d1_ref)
    ls_ref[...] = jnp.concatenate([us_ref[...], uy_ref[...]], axis=1) * dinv


def _layer1_kernel(a0_ref, a1_ref, ls_ref, d0_ref, d1_ref, w_ref, b_ref,
                   h_ref, sum_ref, sq_ref):
    dinv = _dinv_block(d0_ref, d1_ref)
    t = (a0_ref[0] + a1_ref[0] + ls_ref[...]) * dinv
    h = jnp.dot(t, w_ref[...], preferred_element_type=jnp.float32) + b_ref[...]
    h_ref[...] = h
    cs = jnp.sum(h, axis=0, keepdims=True)
    cq = jnp.sum(h * h, axis=0, keepdims=True)
    i = pl.program_id(0)

    @pl.when(i == 0)
    def _():
        sum_ref[...] = cs
        sq_ref[...] = cq

    @pl.when(i > 0)
    def _():
        sum_ref[...] += cs
        sq_ref[...] += cq


def _layer2a_kernel(h_ref, sc_ref, sh_ref, w_ref, d0_ref, d1_ref, xs_ref):
    dinv = _dinv_block(d0_ref, d1_ref)
    h = jnp.maximum(h_ref[...] * sc_ref[...] + sh_ref[...], 0.0)
    xs_ref[...] = jnp.dot(h, w_ref[...],
                          preferred_element_type=jnp.float32) * dinv


def _final_kernel(a0_ref, a1_ref, xs_ref, d0_ref, d1_ref, b_ref,
                  h_ref, sum_ref, sq_ref):
    dinv = _dinv_block(d0_ref, d1_ref)
    h = (a0_ref[0] + a1_ref[0] + xs_ref[...]) * dinv + b_ref[...]
    h_ref[...] = h
    cs = jnp.sum(h, axis=0, keepdims=True)
    cq = jnp.sum(h * h, axis=0, keepdims=True)
    i = pl.program_id(0)

    @pl.when(i == 0)
    def _():
        sum_ref[...] = cs
        sq_ref[...] = cq

    @pl.when(i > 0)
    def _():
        sum_ref[...] += cs
        sq_ref[...] += cq


def _affine_kernel(h_ref, sc_ref, sh_ref, o_ref):
    o_ref[...] = h_ref[...] * sc_ref[...] + sh_ref[...]


def _row_spec(d):
    return pl.BlockSpec((_BM, d), lambda i: (i, 0))


def _agg_specs():
    return [
        pl.BlockSpec((1, _BM, 128), lambda i: (0, i, 0)),
        pl.BlockSpec((1, _BM, 128), lambda i: (1, i, 0)),
    ]


def _full_spec(shape):
    nd = len(shape)
    return pl.BlockSpec(shape, lambda i: (0,) * nd)


def _bn_scale_shift(ssum, ssq, g, be):
    mean = ssum[0] / N
    var = jnp.maximum(ssq[0] / N - mean * mean, 0.0)
    scale = g / jnp.sqrt(var + EPS)
    shift = be - mean * scale
    return scale[None, :], shift[None, :]


def kernel(edge_index, u_S, u_Y, W1, b1, W2, b2, g1, be1, g2, be2):
    src = edge_index[0].astype(jnp.int32)
    dst = edge_index[1].astype(jnp.int32)
    dst_p = jnp.concatenate([dst, jnp.full((_EPAD - E,), _DUMP, jnp.int32)])
    dst_d = dst_p.reshape(_NC, _NS, _NCHD, _CHD)

    nch = _PH_HEAVY * _PCH

    def _split_uneven(flat, pad_val):
        ne_h = min(E, _NE_HEAVY)
        heavy = jnp.concatenate(
            [flat[:ne_h], jnp.full((_NE_HEAVY - ne_h,), pad_val, jnp.int32)]
        ).reshape(_NS, nch, _CHA)
        light = jnp.concatenate(
            [flat[ne_h:], jnp.full((_NE_LIGHT - (E - ne_h),), pad_val,
                                   jnp.int32)]
        ).reshape(_NS, _PH_LIGHT * _PCH, _CHA)
        light = jnp.pad(light, ((0, 0), (0, nch - _PH_LIGHT * _PCH), (0, 0)),
                        constant_values=pad_val)
        return jnp.stack([light, heavy])

    src_a = _split_uneven(src, 0)
    dst_a = _split_uneven(dst, _DUMP)

    ones128 = jnp.ones((_CHD, 128), jnp.float32)
    z128 = jnp.zeros((_SLAB, 128), jnp.float32)

    degp = _sc_degree(dst_d, ones128, z128)

    ls = pl.pallas_call(
        _prescale_kernel,
        grid=(_GRID,),
        in_specs=[_row_spec(64), _row_spec(64)] + _agg_specs(),
        out_specs=_row_spec(128),
        out_shape=jax.ShapeDtypeStruct((N, 128), jnp.float32),
    )(u_S, u_Y, degp, degp)

    agg1 = _sc_aggregate(ls, src_a, dst_a, z128)

    h1, s1, q1 = pl.pallas_call(
        _layer1_kernel,
        grid=(_GRID,),
        in_specs=_agg_specs() + [_row_spec(128)] + _agg_specs()
        + [_full_spec((128, 256)), _full_spec((1, 256))],
        out_specs=[_row_spec(256), _full_spec((1, 256)), _full_spec((1, 256))],
        out_shape=[
            jax.ShapeDtypeStruct((N, 256), jnp.float32),
            jax.ShapeDtypeStruct((1, 256), jnp.float32),
            jax.ShapeDtypeStruct((1, 256), jnp.float32),
        ],
    )(agg1, agg1, ls, degp, degp, W1, b1[None, :])

    sc1, sh1 = _bn_scale_shift(s1, q1, g1, be1)

    xs2 = pl.pallas_call(
        _layer2a_kernel,
        grid=(_GRID,),
        in_specs=[_row_spec(256), _full_spec((1, 256)), _full_spec((1, 256)),
                  _full_spec((256, 128))] + _agg_specs(),
        out_specs=_row_spec(128),
        out_shape=jax.ShapeDtypeStruct((N, 128), jnp.float32),
    )(h1, sc1, sh1, W2, degp, degp)

    agg2 = _sc_aggregate(xs2, src_a, dst_a, z128)

    h2, s2, q2 = pl.pallas_call(
        _final_kernel,
        grid=(_GRID,),
        in_specs=_agg_specs() + [_row_spec(128)] + _agg_specs()
        + [_full_spec((1, 128))],
        out_specs=[_row_spec(128), _full_spec((1, 128)), _full_spec((1, 128))],
        out_shape=[
            jax.ShapeDtypeStruct((N, 128), jnp.float32),
            jax.ShapeDtypeStruct((1, 128), jnp.float32),
            jax.ShapeDtypeStruct((1, 128), jnp.float32),
        ],
    )(agg2, agg2, xs2, degp, degp, b2[None, :])

    sc2, sh2 = _bn_scale_shift(s2, q2, g2, be2)

    out = pl.pallas_call(
        _affine_kernel,
        grid=(_GRID,),
        in_specs=[_row_spec(128), _full_spec((1, 128)), _full_spec((1, 128))],
        out_specs=_row_spec(128),
        out_shape=jax.ShapeDtypeStruct((N, 128), jnp.float32),
    )(h2, sc2, sh2)

    return out


# speedup vs baseline: 1.2514x; 1.2514x over previous
import functools

import jax
import jax.numpy as jnp
from jax import lax
from jax.experimental import pallas as pl
from jax.experimental.pallas import tpu as pltpu
from jax.experimental.pallas import tpu_sc as plsc

N = 10000
E = 320000
EPS = 1e-5

_NC = 2
_NS = 16
_P = 10112
_SLAB = _P // _NS
_DUMP = _P - 1
_EPAD = 327680

_CHD = 128
_NCHD = 80
_NBD = 5

_CHA = 128
_PCH = 20
_NBA = 2
_PH_HEAVY = 7
_PH_LIGHT = 1
_NE_HEAVY = _NS * _PH_HEAVY * _PCH * _CHA
_NE_LIGHT = _NS * _PH_LIGHT * _PCH * _CHA

_BM = 400
_GRID = N // _BM

_mesh = plsc.VectorSubcoreMesh(core_axis_name="c", subcore_axis_name="s")


@functools.partial(
    pl.kernel,
    out_type=jax.ShapeDtypeStruct((_NC, _P, 128), jnp.float32),
    mesh=_mesh,
    scratch_types=[
        pltpu.VMEM_SHARED((_P, 128), jnp.float32),
        pltpu.VMEM((_NCHD, _CHD), jnp.int32),
        pltpu.VMEM((_CHD, 128), jnp.float32),
        pltpu.SemaphoreType.DMA,
    ],
)
def _sc_degree(dst_hbm, ones_hbm, z128_hbm, out_hbm, sh, dsti, onesb, sem):
    c = lax.axis_index("c")
    s = lax.axis_index("s")
    lo = s * _SLAB
    pltpu.sync_copy(dst_hbm.at[c, s], dsti)
    pltpu.sync_copy(z128_hbm, sh.at[pl.ds(lo, _SLAB)])
    pltpu.sync_copy(ones_hbm, onesb)
    plsc.subcore_barrier()

    def body(g, carry):
        base = g * _NBD
        for b in range(_NBD):
            pltpu.async_copy(onesb, sh.at[dsti.at[base + b]], sem, add=True)
        for b in range(_NBD):
            pltpu.make_async_copy(onesb, sh.at[dsti.at[base + b]], sem).wait()
        return carry

    lax.fori_loop(0, _NCHD // _NBD, body, 0)
    plsc.subcore_barrier()
    pltpu.sync_copy(sh.at[pl.ds(lo, _SLAB)], out_hbm.at[c, pl.ds(lo, _SLAB)])


@functools.partial(
    pl.kernel,
    out_type=jax.ShapeDtypeStruct((_NC, _P, 128), jnp.float32),
    mesh=_mesh,
    scratch_types=[
        pltpu.VMEM_SHARED((_P, 128), jnp.float32),
        pltpu.VMEM((_PCH, _CHA), jnp.int32),
        pltpu.VMEM((_PCH, _CHA), jnp.int32),
        pltpu.VMEM((_NBA, _CHA, 128), jnp.float32),
        pltpu.SemaphoreType.DMA((_NBA,)),
        pltpu.SemaphoreType.DMA((_NBA,)),
    ],
)
def _sc_aggregate(x_hbm, src_hbm, dst_hbm, z128_hbm, out_hbm, sh,
                  srci, dsti, rows, sem_g, sem_s):
    c = lax.axis_index("c")
    s = lax.axis_index("s")
    lo = s * _SLAB
    nph = lax.select(c == 0, _PH_LIGHT, _PH_HEAVY)
    pltpu.sync_copy(z128_hbm, sh.at[pl.ds(lo, _SLAB)])
    plsc.subcore_barrier()

    def phase(p, carry):
        @pl.when(p < nph)
        def _():
            pltpu.sync_copy(src_hbm.at[c, s, p], srci)
            pltpu.sync_copy(dst_hbm.at[c, s, p], dsti)

            def rnd(r, cc):
                base = r * _NBA
                for b in range(_NBA):
                    @pl.when(r > 0)
                    def _():
                        pltpu.make_async_copy(
                            rows.at[b], sh.at[dsti.at[base - _NBA + b]],
                            sem_s.at[b]).wait()
                    pltpu.async_copy(x_hbm.at[srci.at[base + b]], rows.at[b],
                                     sem_g.at[b])
                for b in range(_NBA):
                    pltpu.make_async_copy(
                        x_hbm.at[srci.at[base + b]], rows.at[b],
                        sem_g.at[b]).wait()
                    pltpu.async_copy(rows.at[b], sh.at[dsti.at[base + b]],
                                     sem_s.at[b], add=True)
                return cc

            lax.fori_loop(0, _PCH // _NBA, rnd, 0)
            for b in range(_NBA):
                pltpu.make_async_copy(
                    rows.at[b], sh.at[dsti.at[_PCH - _NBA + b]],
                    sem_s.at[b]).wait()

        return carry

    lax.fori_loop(0, _PH_HEAVY, phase, 0)
    plsc.subcore_barrier()
    pltpu.sync_copy(sh.at[pl.ds(lo, _SLAB)], out_hbm.at[c, pl.ds(lo, _SLAB)])


def _dinv_block(d0, d1):
    deg = d0[0, :, :1] + d1[0, :, :1] + 1.0
    return lax.rsqrt(deg)


def _prescale_kernel(us_ref, uy_ref, d0_ref, d1_ref, ls_ref):
    dinv = _dinv_block(d0_ref, d1_ref)
    ls_ref[...] = jnp.concatenate([us_ref[...], uy_ref[...]], axis=1) * dinv


def _layer1_kernel(a0_ref, a1_ref, ls_ref, d0_ref, d1_ref, w_ref, b_ref,
                   h_ref, sum_ref, sq_ref):
    dinv = _dinv_block(d0_ref, d1_ref)
    t = (a0_ref[0] + a1_ref[0] + ls_ref[...]) * dinv
    h = jnp.dot(t, w_ref[...], preferred_element_type=jnp.float32) + b_ref[...]
    h_ref[...] = h
    cs = jnp.sum(h, axis=0, keepdims=True)
    cq = jnp.sum(h * h, axis=0, keepdims=True)
    i = pl.program_id(0)

    @pl.when(i == 0)
    def _():
        sum_ref[...] = cs
        sq_ref[...] = cq

    @pl.when(i > 0)
    def _():
        sum_ref[...] += cs
        sq_ref[...] += cq


def _layer2a_kernel(h_ref, sc_ref, sh_ref, w_ref, d0_ref, d1_ref, xs_ref):
    dinv = _dinv_block(d0_ref, d1_ref)
    h = jnp.maximum(h_ref[...] * sc_ref[...] + sh_ref[...], 0.0)
    xs_ref[...] = jnp.dot(h, w_ref[...],
                          preferred_element_type=jnp.float32) * dinv


def _final_kernel(a0_ref, a1_ref, xs_ref, d0_ref, d1_ref, b_ref,
                  h_ref, sum_ref, sq_ref):
    dinv = _dinv_block(d0_ref, d1_ref)
    h = (a0_ref[0] + a1_ref[0] + xs_ref[...]) * dinv + b_ref[...]
    h_ref[...] = h
    cs = jnp.sum(h, axis=0, keepdims=True)
    cq = jnp.sum(h * h, axis=0, keepdims=True)
    i = pl.program_id(0)

    @pl.when(i == 0)
    def _():
        sum_ref[...] = cs
        sq_ref[...] = cq

    @pl.when(i > 0)
    def _():
        sum_ref[...] += cs
        sq_ref[...] += cq


def _affine_kernel(h_ref, sc_ref, sh_ref, o_ref):
    o_ref[...] = h_ref[...] * sc_ref[...] + sh_ref[...]


def _row_spec(d):
    return pl.BlockSpec((_BM, d), lambda i: (i, 0))


def _agg_specs():
    return [
        pl.BlockSpec((1, _BM, 128), lambda i: (0, i, 0)),
        pl.BlockSpec((1, _BM, 128), lambda i: (1, i, 0)),
    ]


def _full_spec(shape):
    nd = len(shape)
    return pl.BlockSpec(shape, lambda i: (0,) * nd)


def _bn_scale_shift(ssum, ssq, g, be):
    mean = ssum[0] / N
    var = jnp.maximum(ssq[0] / N - mean * mean, 0.0)
    scale = g / jnp.sqrt(var + EPS)
    shift = be - mean * scale
    return scale[None, :], shift[None, :]


def kernel(edge_index, u_S, u_Y, W1, b1, W2, b2, g1, be1, g2, be2):
    src = edge_index[0].astype(jnp.int32)
    dst = edge_index[1].astype(jnp.int32)
    dst_p = jnp.concatenate([dst, jnp.full((_EPAD - E,), _DUMP, jnp.int32)])
    dst_d = dst_p.reshape(_NC, _NS, _NCHD, _CHD)

    def _split_uneven(flat, pad_val):
        ne_h = min(E, _NE_HEAVY)
        heavy = jnp.concatenate(
            [flat[:ne_h], jnp.full((_NE_HEAVY - ne_h,), pad_val, jnp.int32)]
        ).reshape(_NS, _PH_HEAVY, _PCH, _CHA)
        light = jnp.concatenate(
            [flat[ne_h:], jnp.full((_NE_LIGHT - (E - ne_h),), pad_val,
                                   jnp.int32)]
        ).reshape(_NS, _PH_LIGHT, _PCH, _CHA)
        light = jnp.pad(
            light, ((0, 0), (0, _PH_HEAVY - _PH_LIGHT), (0, 0), (0, 0)),
            constant_values=pad_val)
        return jnp.stack([light, heavy])

    src_a = _split_uneven(src, 0)
    dst_a = _split_uneven(dst, _DUMP)

    ones128 = jnp.ones((_CHD, 128), jnp.float32)
    z128 = jnp.zeros((_SLAB, 128), jnp.float32)

    degp = _sc_degree(dst_d, ones128, z128)

    ls = pl.pallas_call(
        _prescale_kernel,
        grid=(_GRID,),
        in_specs=[_row_spec(64), _row_spec(64)] + _agg_specs(),
        out_specs=_row_spec(128),
        out_shape=jax.ShapeDtypeStruct((N, 128), jnp.float32),
    )(u_S, u_Y, degp, degp)

    agg1 = _sc_aggregate(ls, src_a, dst_a, z128)

    h1, s1, q1 = pl.pallas_call(
        _layer1_kernel,
        grid=(_GRID,),
        in_specs=_agg_specs() + [_row_spec(128)] + _agg_specs()
        + [_full_spec((128, 256)), _full_spec((1, 256))],
        out_specs=[_row_spec(256), _full_spec((1, 256)), _full_spec((1, 256))],
        out_shape=[
            jax.ShapeDtypeStruct((N, 256), jnp.float32),
            jax.ShapeDtypeStruct((1, 256), jnp.float32),
            jax.ShapeDtypeStruct((1, 256), jnp.float32),
        ],
    )(agg1, agg1, ls, degp, degp, W1, b1[None, :])

    sc1, sh1 = _bn_scale_shift(s1, q1, g1, be1)

    xs2 = pl.pallas_call(
        _layer2a_kernel,
        grid=(_GRID,),
        in_specs=[_row_spec(256), _full_spec((1, 256)), _full_spec((1, 256)),
                  _full_spec((256, 128))] + _agg_specs(),
        out_specs=_row_spec(128),
        out_shape=jax.ShapeDtypeStruct((N, 128), jnp.float32),
    )(h1, sc1, sh1, W2, degp, degp)

    agg2 = _sc_aggregate(xs2, src_a, dst_a, z128)

    h2, s2, q2 = pl.pallas_call(
        _final_kernel,
        grid=(_GRID,),
        in_specs=_agg_specs() + [_row_spec(128)] + _agg_specs()
        + [_full_spec((1, 128))],
        out_specs=[_row_spec(128), _full_spec((1, 128)), _full_spec((1, 128))],
        out_shape=[
            jax.ShapeDtypeStruct((N, 128), jnp.float32),
            jax.ShapeDtypeStruct((1, 128), jnp.float32),
            jax.ShapeDtypeStruct((1, 128), jnp.float32),
        ],
    )(agg2, agg2, xs2, degp, degp, b2[None, :])

    sc2, sh2 = _bn_scale_shift(s2, q2, g2, be2)

    out = pl.pallas_call(
        _affine_kernel,
        grid=(_GRID,),
        in_specs=[_row_spec(128), _full_spec((1, 128)), _full_spec((1, 128))],
        out_specs=_row_spec(128),
        out_shape=jax.ShapeDtypeStruct((N, 128), jnp.float32),
    )(h2, sc2, sh2)

    return out
